# baseline (speedup 1.0000x reference)
import jax
import jax.numpy as jnp
from jax.experimental import pallas as pl
from jax.experimental.pallas import tpu as pltpu

_K = 64
_D = 128
_BLK = 2048


def _vq_body(x_ref, emb_ref, esq_ref, q_ref, loss_ref):
    i = pl.program_id(0)
    x = x_ref[...]
    emb = emb_ref[...]
    esq = esq_ref[...]
    scores = jax.lax.dot_general(x, emb, (((1,), (1,)), ((), ())),
                                 preferred_element_type=jnp.float32)
    xsq = jnp.sum(x * x, axis=1, keepdims=True)
    dist = (xsq + esq) - 2.0 * scores
    min_val = jnp.min(dist, axis=1, keepdims=True)
    iota = jax.lax.broadcasted_iota(jnp.int32, dist.shape, 1)
    masked = jnp.where(dist <= min_val, iota, _K)
    idx = jnp.min(masked, axis=1, keepdims=True)
    onehot = (iota == idx).astype(jnp.float32)
    q = jax.lax.dot_general(onehot, emb, (((1,), (0,)), ((), ())),
                            preferred_element_type=jnp.float32)
    q_ref[...] = q
    diff = q - x
    part = jnp.sum(diff * diff)

    @pl.when(i == 0)
    def _init():
        loss_ref[0, 0] = 0.0

    loss_ref[0, 0] += part


def kernel(inputs, embeddings):
    n, d = inputs.shape
    esq = jnp.sum(embeddings ** 2, axis=1).reshape(1, _K)
    grid = (n // _BLK,)
    q, loss = pl.pallas_call(
        _vq_body,
        grid=grid,
        in_specs=[
            pl.BlockSpec((_BLK, d), lambda i: (i, 0)),
            pl.BlockSpec((_K, d), lambda i: (0, 0)),
            pl.BlockSpec((1, _K), lambda i: (0, 0)),
        ],
        out_specs=[
            pl.BlockSpec((_BLK, d), lambda i: (i, 0)),
            pl.BlockSpec(memory_space=pltpu.SMEM),
        ],
        out_shape=[
            jax.ShapeDtypeStruct((n, d), jnp.float32),
            jax.ShapeDtypeStruct((1, 1), jnp.float32),
        ],
    )(inputs, embeddings, esq)
    vq_loss = (2.0 / (n * d)) * loss[0, 0]
    return (q, vq_loss)


# speedup vs baseline: 1.3299x; 1.3299x over previous
import jax
import jax.numpy as jnp
from jax.experimental import pallas as pl
from jax.experimental.pallas import tpu as pltpu

_K = 64
_D = 128
_BLK = 8192


def _vq_body(x_ref, emb_ref, esq_ref, q_ref, loss_ref):
    i = pl.program_id(0)
    x = x_ref[...]
    emb = emb_ref[...]
    esq = esq_ref[...]
    scores = jax.lax.dot_general(x, emb, (((1,), (1,)), ((), ())),
                                 preferred_element_type=jnp.float32)
    xsq = jnp.sum(x * x, axis=1, keepdims=True)
    dist = (xsq + esq) - 2.0 * scores
    min_val = jnp.min(dist, axis=1, keepdims=True)
    iota = jax.lax.broadcasted_iota(jnp.int32, dist.shape, 1)
    masked = jnp.where(dist <= min_val, iota, _K)
    idx = jnp.min(masked, axis=1, keepdims=True)
    onehot = (iota == idx).astype(jnp.float32)
    q = jax.lax.dot_general(onehot, emb, (((1,), (0,)), ((), ())),
                            preferred_element_type=jnp.float32)
    q_ref[...] = q
    diff = q - x
    part = jnp.sum(diff * diff)

    @pl.when(i == 0)
    def _init():
        loss_ref[0, 0] = 0.0

    loss_ref[0, 0] += part


def kernel(inputs, embeddings):
    n, d = inputs.shape
    esq = jnp.sum(embeddings ** 2, axis=1).reshape(1, _K)
    grid = (n // _BLK,)
    q, loss = pl.pallas_call(
        _vq_body,
        grid=grid,
        in_specs=[
            pl.BlockSpec((_BLK, d), lambda i: (i, 0)),
            pl.BlockSpec((_K, d), lambda i: (0, 0)),
            pl.BlockSpec((1, _K), lambda i: (0, 0)),
        ],
        out_specs=[
            pl.BlockSpec((_BLK, d), lambda i: (i, 0)),
            pl.BlockSpec(memory_space=pltpu.SMEM),
        ],
        out_shape=[
            jax.ShapeDtypeStruct((n, d), jnp.float32),
            jax.ShapeDtypeStruct((1, 1), jnp.float32),
        ],
    )(inputs, embeddings, esq)
    vq_loss = (2.0 / (n * d)) * loss[0, 0]
    return (q, vq_loss)


# speedup vs baseline: 1.3690x; 1.0294x over previous
import jax
import jax.numpy as jnp
from jax.experimental import pallas as pl
from jax.experimental.pallas import tpu as pltpu

_K = 64
_D = 128
_BLK = 16384


def _vq_body(x_ref, emb_ref, esq_ref, q_ref, loss_ref):
    i = pl.program_id(0)
    x = x_ref[...]
    emb = emb_ref[...]
    esq = esq_ref[...]
    scores = jax.lax.dot_general(x, emb, (((1,), (1,)), ((), ())),
                                 preferred_element_type=jnp.float32)
    xsq = jnp.sum(x * x, axis=1, keepdims=True)
    dist = (xsq + esq) - 2.0 * scores
    min_val = jnp.min(dist, axis=1, keepdims=True)
    iota = jax.lax.broadcasted_iota(jnp.int32, dist.shape, 1)
    masked = jnp.where(dist <= min_val, iota, _K)
    idx = jnp.min(masked, axis=1, keepdims=True)
    onehot = (iota == idx).astype(jnp.float32)
    q = jax.lax.dot_general(onehot, emb, (((1,), (0,)), ((), ())),
                            preferred_element_type=jnp.float32)
    q_ref[...] = q
    diff = q - x
    part = jnp.sum(diff * diff)

    @pl.when(i == 0)
    def _init():
        loss_ref[0, 0] = 0.0

    loss_ref[0, 0] += part


def kernel(inputs, embeddings):
    n, d = inputs.shape
    esq = jnp.sum(embeddings ** 2, axis=1).reshape(1, _K)
    grid = (n // _BLK,)
    q, loss = pl.pallas_call(
        _vq_body,
        grid=grid,
        in_specs=[
            pl.BlockSpec((_BLK, d), lambda i: (i, 0)),
            pl.BlockSpec((_K, d), lambda i: (0, 0)),
            pl.BlockSpec((1, _K), lambda i: (0, 0)),
        ],
        out_specs=[
            pl.BlockSpec((_BLK, d), lambda i: (i, 0)),
            pl.BlockSpec(memory_space=pltpu.SMEM),
        ],
        out_shape=[
            jax.ShapeDtypeStruct((n, d), jnp.float32),
            jax.ShapeDtypeStruct((1, 1), jnp.float32),
        ],
    )(inputs, embeddings, esq)
    vq_loss = (2.0 / (n * d)) * loss[0, 0]
    return (q, vq_loss)


# speedup vs baseline: 1.8268x; 1.3344x over previous
import jax
import jax.numpy as jnp
from jax.experimental import pallas as pl
from jax.experimental.pallas import tpu as pltpu

_K = 64
_D = 128
_BLK = 16384


def _vq_body(x_ref, emb_ref, esq_ref, q_ref, loss_ref):
    i = pl.program_id(0)
    x = x_ref[...]
    emb = emb_ref[...]
    esq = esq_ref[...]
    scores2 = jax.lax.dot_general(x, 2.0 * emb, (((1,), (1,)), ((), ())),
                                  preferred_element_type=jnp.float32)
    xsq = jnp.sum(x * x, axis=1, keepdims=True)
    dist = (xsq + esq) - scores2
    min_val = jnp.min(dist, axis=1, keepdims=True)
    iota = jax.lax.broadcasted_iota(jnp.int32, dist.shape, 1).astype(jnp.float32)
    masked = jnp.where(dist <= min_val, iota, float(_K))
    idx = jnp.min(masked, axis=1, keepdims=True)
    onehot = jnp.where(iota == idx, 1.0, 0.0)
    q = jax.lax.dot_general(onehot, emb, (((1,), (0,)), ((), ())),
                            preferred_element_type=jnp.float32)
    q_ref[...] = q
    part = jnp.sum(min_val)

    @pl.when(i == 0)
    def _init():
        loss_ref[0, 0] = 0.0

    loss_ref[0, 0] += part


def kernel(inputs, embeddings):
    n, d = inputs.shape
    esq = jnp.sum(embeddings ** 2, axis=1).reshape(1, _K)
    grid = (n // _BLK,)
    q, loss = pl.pallas_call(
        _vq_body,
        grid=grid,
        in_specs=[
            pl.BlockSpec((_BLK, d), lambda i: (i, 0)),
            pl.BlockSpec((_K, d), lambda i: (0, 0)),
            pl.BlockSpec((1, _K), lambda i: (0, 0)),
        ],
        out_specs=[
            pl.BlockSpec((_BLK, d), lambda i: (i, 0)),
            pl.BlockSpec(memory_space=pltpu.SMEM),
        ],
        out_shape=[
            jax.ShapeDtypeStruct((n, d), jnp.float32),
            jax.ShapeDtypeStruct((1, 1), jnp.float32),
        ],
    )(inputs, embeddings, esq)
    vq_loss = (2.0 / (n * d)) * loss[0, 0]
    return (q, vq_loss)
